# Initial kernel scaffold; baseline (speedup 1.0000x reference)
import jax
import jax.numpy as jnp
from jax.experimental import pallas as pl


def kernel(input_ids, prompt_ids, word_emb, prompt_emb, token_type_emb, pos_emb, ln_gamma, ln_beta):
    raise NotImplementedError("write your pallas kernel here")


# speedup vs baseline: 1.6174x; 1.6174x over previous
import functools

import jax
import jax.numpy as jnp
from jax import lax
from jax.experimental import pallas as pl
from jax.experimental.pallas import tpu as pltpu
from jax.experimental.pallas import tpu_sc as plsc

VOCAB = 30522
HID = 768
PVOCAB = 100
PLEN = 20
B = 32
S = 512
EPS = 1e-12

NW = 32
ROWS_PER_W = S
CHUNK = 64
NCHUNK = ROWS_PER_W // CHUNK
PPAD = 24


def _sc_gather(word_emb, prompt_emb, wids_flat, pids_pad):
    mesh = plsc.VectorSubcoreMesh(core_axis_name="c", subcore_axis_name="s")

    @functools.partial(
        pl.kernel,
        out_type=(
            jax.ShapeDtypeStruct((B * S, HID), jnp.float32),
            jax.ShapeDtypeStruct((B * PPAD, HID), jnp.float32),
        ),
        mesh=mesh,
        scratch_types=[
            pltpu.VMEM((ROWS_PER_W,), jnp.int32),
            pltpu.VMEM((2, CHUNK, HID), jnp.float32),
            pltpu.VMEM((PPAD,), jnp.int32),
            pltpu.VMEM((PPAD, HID), jnp.float32),
            pltpu.SemaphoreType.DMA,
            pltpu.SemaphoreType.DMA,
            pltpu.SemaphoreType.DMA,
        ],
    )
    def k(word_hbm, pemb_hbm, wids_hbm, pids_hbm, out_hbm, pout_hbm,
          idx_v, rows_v, pidx_v, prows_v, sem0, sem1, psem):
        sems = (sem0, sem1)
        w = lax.axis_index("s") * 2 + lax.axis_index("c")
        base = w * ROWS_PER_W
        pltpu.sync_copy(wids_hbm.at[pl.ds(base, ROWS_PER_W)], idx_v)
        pltpu.sync_copy(pids_hbm.at[pl.ds(w * PPAD, PPAD)], pidx_v)

        pc = pltpu.make_async_copy(pemb_hbm.at[pidx_v], prows_v, psem)
        pc.start()
        copies = [None, None]
        copies[0] = pltpu.make_async_copy(
            word_hbm.at[idx_v.at[pl.ds(0, CHUNK)]], rows_v.at[0], sems[0])
        copies[0].start()
        for c in range(NCHUNK):
            buf = c % 2
            if c + 1 < NCHUNK:
                nbuf = (c + 1) % 2
                copies[nbuf] = pltpu.make_async_copy(
                    word_hbm.at[idx_v.at[pl.ds((c + 1) * CHUNK, CHUNK)]],
                    rows_v.at[nbuf], sems[nbuf])
                copies[nbuf].start()
            copies[buf].wait()
            pltpu.sync_copy(rows_v.at[buf],
                            out_hbm.at[pl.ds(base + c * CHUNK, CHUNK)])
        pc.wait()
        pltpu.sync_copy(prows_v, pout_hbm.at[pl.ds(w * PPAD, PPAD)])

    return k(word_emb, prompt_emb, wids_flat, pids_pad)


def _tc_ln_body(g_ref, pg_ref, pos_ref, type_ref, gamma_ref, beta_ref, o_ref):
    sblk = pl.program_id(1)
    g = g_ref[...]
    pg = jnp.pad(pg_ref[...][:PLEN], ((1, g.shape[0] - PLEN - 1), (0, 0)))
    row = lax.broadcasted_iota(jnp.int32, (g.shape[0], 1), 0)
    mask = (row >= 1) & (row <= PLEN) & (sblk == 0)
    x = jnp.where(mask, pg, g) + pos_ref[...] + type_ref[...]
    mu = jnp.mean(x, axis=-1, keepdims=True)
    d = x - mu
    var = jnp.mean(d * d, axis=-1, keepdims=True)
    o_ref[...] = d * lax.rsqrt(var + EPS) * gamma_ref[...] + beta_ref[...]


def _tc_ln(gathered, pgath, pos_emb, type_row, gamma, beta):
    SB = 128
    grid = (B, S // SB)
    return pl.pallas_call(
        _tc_ln_body,
        grid=grid,
        in_specs=[
            pl.BlockSpec((SB, HID), lambda b, s: (b * (S // SB) + s, 0)),
            pl.BlockSpec((PPAD, HID), lambda b, s: (b, 0)),
            pl.BlockSpec((SB, HID), lambda b, s: (s, 0)),
            pl.BlockSpec((1, HID), lambda b, s: (0, 0)),
            pl.BlockSpec((1, HID), lambda b, s: (0, 0)),
            pl.BlockSpec((1, HID), lambda b, s: (0, 0)),
        ],
        out_specs=pl.BlockSpec((SB, HID), lambda b, s: (b * (S // SB) + s, 0)),
        out_shape=jax.ShapeDtypeStruct((B * S, HID), jnp.float32),
    )(gathered, pgath, pos_emb, type_row, gamma, beta)


def kernel(input_ids, prompt_ids, word_emb, prompt_emb, token_type_emb,
           pos_emb, ln_gamma, ln_beta):
    wids_flat = input_ids.reshape(-1)
    pids_pad = jnp.pad(prompt_ids, ((0, 0), (0, PPAD - PLEN))).reshape(-1)
    gathered, pgath = _sc_gather(word_emb, prompt_emb, wids_flat, pids_pad)
    out = _tc_ln(gathered, pgath, pos_emb, token_type_emb[:1],
                 ln_gamma.reshape(1, HID), ln_beta.reshape(1, HID))
    return out.reshape(B, S, HID)


# speedup vs baseline: 2.5131x; 1.5538x over previous
import functools

import jax
import jax.numpy as jnp
from jax import lax
from jax.experimental import pallas as pl
from jax.experimental.pallas import tpu as pltpu
from jax.experimental.pallas import tpu_sc as plsc

VOCAB = 30522
HID = 768
PVOCAB = 100
PLEN = 20
B = 32
S = 512
EPS = 1e-12

NW = 32
ROWS_PER_W = S
CHUNK = 64
NCHUNK = ROWS_PER_W // CHUNK
PPAD = 24


def _sc_gather(word_emb, prompt_emb, wids_flat, pids_pad):
    mesh = plsc.VectorSubcoreMesh(core_axis_name="c", subcore_axis_name="s")

    @functools.partial(
        pl.kernel,
        out_type=(
            jax.ShapeDtypeStruct((B * S, HID), jnp.float32),
            jax.ShapeDtypeStruct((B * PPAD, HID), jnp.float32),
        ),
        mesh=mesh,
        scratch_types=[
            pltpu.VMEM((ROWS_PER_W,), jnp.int32),
            pltpu.VMEM((2, CHUNK, HID), jnp.float32),
            pltpu.VMEM((PPAD,), jnp.int32),
            pltpu.VMEM((PPAD, HID), jnp.float32),
            pltpu.SemaphoreType.DMA,
            pltpu.SemaphoreType.DMA,
            pltpu.SemaphoreType.DMA,
        ],
    )
    def k(word_hbm, pemb_hbm, wids_hbm, pids_hbm, out_hbm, pout_hbm,
          idx_v, rows_v, pidx_v, prows_v, sem0, sem1, psem):
        sems = (sem0, sem1)
        w = lax.axis_index("s") * 2 + lax.axis_index("c")
        base = w * ROWS_PER_W
        pltpu.sync_copy(wids_hbm.at[pl.ds(base, ROWS_PER_W)], idx_v)
        pltpu.sync_copy(pids_hbm.at[pl.ds(w * PPAD, PPAD)], pidx_v)

        pc = pltpu.make_async_copy(pemb_hbm.at[pidx_v], prows_v, psem)
        pc.start()
        copies = [None, None]
        copies[0] = pltpu.make_async_copy(
            word_hbm.at[idx_v.at[pl.ds(0, CHUNK)]], rows_v.at[0], sems[0])
        copies[0].start()
        for c in range(NCHUNK):
            buf = c % 2
            if c + 1 < NCHUNK:
                nbuf = (c + 1) % 2
                copies[nbuf] = pltpu.make_async_copy(
                    word_hbm.at[idx_v.at[pl.ds((c + 1) * CHUNK, CHUNK)]],
                    rows_v.at[nbuf], sems[nbuf])
                copies[nbuf].start()
            copies[buf].wait()
            pltpu.sync_copy(rows_v.at[buf],
                            out_hbm.at[pl.ds(base + c * CHUNK, CHUNK)])
        pc.wait()
        pltpu.sync_copy(prows_v, pout_hbm.at[pl.ds(w * PPAD, PPAD)])

    return k(word_emb, prompt_emb, wids_flat, pids_pad)


def _tc_ln_body(g_ref, pg_ref, pos_ref, type_ref, gamma_ref, beta_ref, o_ref):
    sblk = pl.program_id(0)
    g = g_ref[...]
    pg = jnp.pad(pg_ref[...][:PLEN], ((1, g.shape[0] - PLEN - 1), (0, 0)))
    row = lax.broadcasted_iota(jnp.int32, (g.shape[0], 1), 0)
    mask = (row >= 1) & (row <= PLEN) & (sblk == 0)
    x = jnp.where(mask, pg, g) + pos_ref[...] + type_ref[...]
    mu = jnp.mean(x, axis=-1, keepdims=True)
    d = x - mu
    var = jnp.mean(d * d, axis=-1, keepdims=True)
    o_ref[...] = d * lax.rsqrt(var + EPS) * gamma_ref[...] + beta_ref[...]


def _tc_ln(gathered, pgath, pos_emb, type_row, gamma, beta):
    SB = 512
    nsb = S // SB
    grid = (nsb, B)
    return pl.pallas_call(
        _tc_ln_body,
        grid=grid,
        in_specs=[
            pl.BlockSpec((SB, HID), lambda s, b: (b * nsb + s, 0)),
            pl.BlockSpec((PPAD, HID), lambda s, b: (b, 0)),
            pl.BlockSpec((SB, HID), lambda s, b: (s, 0)),
            pl.BlockSpec((1, HID), lambda s, b: (0, 0)),
            pl.BlockSpec((1, HID), lambda s, b: (0, 0)),
            pl.BlockSpec((1, HID), lambda s, b: (0, 0)),
        ],
        out_specs=pl.BlockSpec((SB, HID), lambda s, b: (b * nsb + s, 0)),
        out_shape=jax.ShapeDtypeStruct((B * S, HID), jnp.float32),
    )(gathered, pgath, pos_emb, type_row, gamma, beta)


def kernel(input_ids, prompt_ids, word_emb, prompt_emb, token_type_emb,
           pos_emb, ln_gamma, ln_beta):
    wids_flat = input_ids.reshape(-1)
    pids_pad = jnp.pad(prompt_ids, ((0, 0), (0, PPAD - PLEN))).reshape(-1)
    gathered, pgath = _sc_gather(word_emb, prompt_emb, wids_flat, pids_pad)
    out = _tc_ln(gathered, pgath, pos_emb, token_type_emb[:1],
                 ln_gamma.reshape(1, HID), ln_beta.reshape(1, HID))
    return out.reshape(B, S, HID)
